# Initial kernel scaffold; baseline (speedup 1.0000x reference)
import jax
import jax.numpy as jnp
from jax.experimental import pallas as pl


def kernel(x, edge_index, batch, Wrel1, brel1, Wroot1, pw1, Wrel2, brel2, Wroot2, pw2, Wrel3, brel3, Wroot3, pw3, Wl1, bl1, Wl2, bl2, Wl3, bl3, g1, be1, g2, be2):
    raise NotImplementedError("write your pallas kernel here")


# speedup vs baseline: 3.3179x; 3.3179x over previous
import functools
import math

import jax
import jax.numpy as jnp
from jax.experimental import pallas as pl
from jax.experimental.pallas import tpu as pltpu

N = 50000
E = 800000
H = 128
RATIO = 0.8
EPS = 1e-5


def _ordkey(s):
    b = jax.lax.bitcast_convert_type(s, jnp.uint32)
    return jnp.where(s >= 0, b | jnp.uint32(0x80000000), ~b)


def _kth_thresh(key, k):
    t = jnp.uint32(0)
    for b in range(31, -1, -1):
        cand = t | jnp.uint32(1 << b)
        c = jnp.sum((key >= cand).astype(jnp.int32))
        t = jnp.where(c >= k, cand, t)
    return t


def _head_kernel(z_ref, wl1_ref, bl1_ref, g1_ref, be1_ref, wl2_ref, bl2_ref,
                 g2_ref, be2_ref, wl3_ref, bl3_ref, out_ref):
    z = z_ref[...]
    s1 = 1.0 / math.sqrt(1.0 + EPS)
    fc1 = jnp.maximum(
        (jnp.dot(z, wl1_ref[...], preferred_element_type=jnp.float32)
         + bl1_ref[...]) * s1 * g1_ref[...] + be1_ref[...], 0.0)
    fc2 = jnp.maximum(
        (jnp.dot(fc1, wl2_ref[...], preferred_element_type=jnp.float32)
         + bl2_ref[...]) * s1 * g2_ref[...] + be2_ref[...], 0.0)
    logits = (jnp.dot(fc2, wl3_ref[...], preferred_element_type=jnp.float32)
              + bl3_ref[...])
    m = jnp.max(logits, axis=1, keepdims=True)
    e = jnp.exp(logits - m)
    out_ref[...] = e / jnp.sum(e, axis=1, keepdims=True)


def _head(z, Wl1, bl1, g1, be1, Wl2, bl2, g2, be2, Wl3, bl3):
    zp = jnp.zeros((8, 256), jnp.float32).at[0].set(z)
    wl2p = jnp.zeros((128, 128), jnp.float32).at[:, :64].set(Wl2)
    bl2p = jnp.full((1, 128), -1e30, jnp.float32).at[0, :64].set(bl2)
    g2p = jnp.zeros((1, 128), jnp.float32).at[0, :64].set(g2)
    be2p = jnp.zeros((1, 128), jnp.float32).at[0, :64].set(be2)
    wl3p = jnp.zeros((128, 128), jnp.float32).at[:64, :2].set(Wl3)
    bl3p = jnp.full((1, 128), -1e30, jnp.float32).at[0, :2].set(bl3)
    out = pl.pallas_call(
        _head_kernel,
        out_shape=jax.ShapeDtypeStruct((8, 128), jnp.float32),
    )(zp, Wl1, bl1[None, :], g1[None, :], be1[None, :], wl2p, bl2p, g2p,
      be2p, wl3p, bl3p)
    return out[0:1, 0:2]


def kernel(x, edge_index, batch, Wrel1, brel1, Wroot1, pw1, Wrel2, brel2,
           Wroot2, pw2, Wrel3, brel3, Wroot3, pw3, Wl1, bl1, Wl2, bl2, Wl3,
           bl3, g1, be1, g2, be2):
    src, dst = edge_index[0], edge_index[1]
    alive = jnp.ones((N,), jnp.float32)
    p = x
    reads = []
    n_alive = N
    for (Wrel, brel, Wroot, pw) in ((Wrel1, brel1, Wroot1, pw1),
                                    (Wrel2, brel2, Wroot2, pw2),
                                    (Wrel3, brel3, Wroot3, pw3)):
        agg = jax.ops.segment_sum(p[src], dst, num_segments=N)
        h = jnp.maximum(agg @ Wrel + brel + p @ Wroot, 0.0)
        s = (h @ pw) / (jnp.linalg.norm(pw) + 1e-16)
        key = jnp.where(alive > 0, _ordkey(s), jnp.uint32(0))
        k = int(math.ceil(RATIO * n_alive))
        t = _kth_thresh(key, k)
        kept = (key >= t).astype(jnp.float32)
        p = h * (jnp.tanh(s) * kept)[:, None]
        mx = jnp.max(jnp.where(kept[:, None] > 0, p, -jnp.inf), axis=0)
        mean = jnp.sum(p, axis=0) / k
        reads.append(jnp.concatenate([mx, mean]))
        alive = kept
        n_alive = k
    z = reads[0] + reads[1] + reads[2]
    return _head(z, Wl1, bl1, g1, be1, Wl2, bl2, g2, be2, Wl3, bl3)


# speedup vs baseline: 16.2053x; 4.8843x over previous
import functools
import math

import jax
import jax.numpy as jnp
from jax import lax
from jax.experimental import pallas as pl
from jax.experimental.pallas import tpu as pltpu
from jax.experimental.pallas import tpu_sc as plsc

N = 50000
E = 800000
H = 128
RATIO = 0.8
EPS = 1e-5

NR = 50176
NB = 4
RNG = 12544
ACCROWS = 12672
WIN = 1024
CAP_WB = 26880
EPW = 25600
E2 = 32 * EPW
GRP = 1024
NGRP = EPW // GRP
PAD_SRC = 50000
PAD_DST = 12544

_mesh = plsc.VectorSubcoreMesh(core_axis_name="c", subcore_axis_name="s")


def _iota16():
    return lax.iota(jnp.int32, 16)


def _bucket_body(src_hbm, dst_hbm, bsrc_hbm, bdst_hbm, nwin_hbm,
                 gsrc, gdst, stages, pad_src_buf, pad_dst_buf, nw_st):
    c = lax.axis_index("c")
    s = lax.axis_index("s")
    w = 2 * s + c
    it = _iota16()

    def fill_pad(i, _):
        pad_src_buf[pl.ds(i * 16, 16)] = PAD_SRC + it
        pad_dst_buf[pl.ds(i * 16, 16)] = PAD_DST + it
        return 0
    lax.fori_loop(0, GRP // 16, fill_pad, 0)

    offs = [jnp.int32(0)] * NB
    for g in range(NGRP):
        base = pl.multiple_of(w * EPW + g * GRP, GRP)
        pltpu.sync_copy(src_hbm.at[pl.ds(base, GRP)], gsrc)
        pltpu.sync_copy(dst_hbm.at[pl.ds(base, GRP)], gdst)

        def vbody(v, locs):
            s16 = gsrc[pl.ds(v * 16, 16)]
            d16 = gdst[pl.ds(v * 16, 16)]
            new = []
            for b in range(NB):
                lo = b * RNG
                m = jnp.logical_and(d16 >= lo, d16 < lo + RNG)
                pos = plsc.cumsum(m.astype(jnp.int32))
                idx = locs[b] + pos - 1
                plsc.store_scatter(stages[2 * b], [idx], s16, mask=m)
                plsc.store_scatter(stages[2 * b + 1], [idx], d16 - lo, mask=m)
                new.append(locs[b] + jnp.max(pos))
            return tuple(new)

        locs = lax.fori_loop(0, GRP // 16, vbody, tuple(jnp.int32(0)
                                                        for _ in range(NB)))
        for b in range(NB):
            l = locs[b]
            r = (8 - lax.rem(l, 8)) % 8
            m = it < r
            plsc.store_scatter(stages[2 * b], [l + it], PAD_SRC + it, mask=m)
            plsc.store_scatter(stages[2 * b + 1], [l + it], PAD_DST + it,
                               mask=m)
            l = l + r
            segbase = (w * NB + b) * CAP_WB
            fo = pl.multiple_of(segbase + offs[b], 8)
            pltpu.sync_copy(stages[2 * b].at[pl.ds(0, GRP)],
                            bsrc_hbm.at[pl.ds(fo, GRP)])
            pltpu.sync_copy(stages[2 * b + 1].at[pl.ds(0, GRP)],
                            bdst_hbm.at[pl.ds(fo, GRP)])
            offs[b] = offs[b] + l

    nwv = jnp.zeros((16,), jnp.int32)
    for b in range(NB):
        segbase = (w * NB + b) * CAP_WB
        fo = pl.multiple_of(segbase + offs[b], 8)
        pltpu.sync_copy(pad_src_buf, bsrc_hbm.at[pl.ds(fo, GRP)])
        pltpu.sync_copy(pad_dst_buf, bdst_hbm.at[pl.ds(fo, GRP)])
        nw = (offs[b] + (WIN - 1)) // WIN
        nwv = jnp.where(it == b, nw, nwv)
    nw_st[...] = nwv
    pltpu.sync_copy(nw_st.at[pl.ds(0, 8)],
                    nwin_hbm.at[pl.ds(pl.multiple_of(w * 8, 8), 8)])


_bucket_call = pl.kernel(
    _bucket_body,
    out_type=[
        jax.ShapeDtypeStruct((128 * CAP_WB,), jnp.int32),
        jax.ShapeDtypeStruct((128 * CAP_WB,), jnp.int32),
        jax.ShapeDtypeStruct((256,), jnp.int32),
    ],
    mesh=_mesh,
    scratch_types=[
        pltpu.VMEM((GRP,), jnp.int32),
        pltpu.VMEM((GRP,), jnp.int32),
        [pltpu.VMEM((GRP + 16,), jnp.int32) for _ in range(2 * NB)],
        pltpu.VMEM((GRP,), jnp.int32),
        pltpu.VMEM((GRP,), jnp.int32),
        pltpu.VMEM((16,), jnp.int32),
    ],
    compiler_params=pltpu.CompilerParams(needs_layout_passes=False),
)


def _conv_body(p_hbm, bsrc_hbm, bdst_hbm, nwin_hbm, out_hbm,
               acc, idx_v, dst_v, rows_v, zero_v, nw_v, sem, *, D):
    c = lax.axis_index("c")
    s = lax.axis_index("s")
    it = _iota16()

    pltpu.sync_copy(p_hbm.at[pl.ds(PAD_SRC + 48, 8)], zero_v)

    def zero_acc(j, _):
        pltpu.sync_copy(zero_v, acc.at[pl.ds(
            pl.multiple_of(s * 792 + j * 8, 8), 8)])
        return 0

    lax.fori_loop(0, 99, zero_acc, 0)
    pltpu.sync_copy(nwin_hbm.at[pl.ds(pl.multiple_of(s * 16, 16), 16)], nw_v)
    plsc.subcore_barrier()

    for phase in range(2):
        b = 2 * c + phase
        for dw in range(2):
            w = 2 * s + dw
            lane = dw * 8 + b
            nw = jnp.max(jnp.where(it == lane, nw_v[...], 0))
            segrow = w * NB + b

            def win_body(win, _):
                wo = pl.multiple_of(win * 8, 8)
                pltpu.sync_copy(bsrc_hbm.at[segrow, pl.ds(wo, 8)], idx_v)
                pltpu.sync_copy(bdst_hbm.at[segrow, pl.ds(wo, 8)], dst_v)
                for j in range(8):
                    pltpu.async_copy(p_hbm.at[idx_v.at[j]],
                                     rows_v, sem).wait()
                    pltpu.sync_copy(rows_v, acc.at[dst_v.at[j]], add=True)
                return 0

            lax.fori_loop(0, nw, win_body, 0)
        plsc.subcore_barrier()
        pltpu.sync_copy(acc.at[pl.ds(s * 784, 784)],
                        out_hbm.at[pl.ds(b * RNG + s * 784, 784)])

        if phase == 0:
            plsc.subcore_barrier()
            lax.fori_loop(0, 99, zero_acc, 0)
            plsc.subcore_barrier()


def _make_conv(D):
    return pl.kernel(
        functools.partial(_conv_body, D=D),
        out_type=jax.ShapeDtypeStruct((NR, D), jnp.float32),
        mesh=_mesh,
        scratch_types=[
            pltpu.VMEM_SHARED((ACCROWS, D), jnp.float32),
            pltpu.VMEM((8, 128), jnp.int32),
            pltpu.VMEM((8, 128), jnp.int32),
            pltpu.VMEM((128, D), jnp.float32),
            pltpu.VMEM((8, D), jnp.float32),
            pltpu.VMEM((16,), jnp.int32),
            pltpu.SemaphoreType.DMA,
        ],
        compiler_params=pltpu.CompilerParams(
            needs_layout_passes=False,
            use_tc_tiling_on_sc=(D == 128)),
    )


_conv16 = _make_conv(16)
_conv128 = _make_conv(128)


_RB = 3584
_NBLK = NR // _RB


def _ordkey(s):
    b = lax.bitcast_convert_type(s, jnp.uint32)
    return jnp.where(s >= 0, b | jnp.uint32(0x80000000), ~b)


def _dense_body(agg_ref, p_ref, wrel_ref, brel_ref, wroot_ref, pwc_ref,
                h_ref, s_ref):
    h = jnp.maximum(
        jnp.dot(agg_ref[...], wrel_ref[...],
                preferred_element_type=jnp.float32) + brel_ref[...]
        + jnp.dot(p_ref[...], wroot_ref[...],
                  preferred_element_type=jnp.float32), 0.0)
    h_ref[...] = h
    pwc = pwc_ref[...]
    nrm = jnp.sqrt(jnp.sum(pwc * pwc))
    s_ref[...] = jnp.dot(h, pwc, preferred_element_type=jnp.float32) \
        / (nrm + 1e-16)


def _dense(agg, p, wrel, brel, wroot, pwc, DP):
    return pl.pallas_call(
        _dense_body,
        grid=(_NBLK,),
        in_specs=[
            pl.BlockSpec((_RB, DP), lambda i: (i, 0)),
            pl.BlockSpec((_RB, DP), lambda i: (i, 0)),
            pl.BlockSpec((DP, H), lambda i: (0, 0)),
            pl.BlockSpec((1, H), lambda i: (0, 0)),
            pl.BlockSpec((DP, H), lambda i: (0, 0)),
            pl.BlockSpec((H, 1), lambda i: (0, 0)),
        ],
        out_specs=[
            pl.BlockSpec((_RB, H), lambda i: (i, 0)),
            pl.BlockSpec((_RB, 1), lambda i: (i, 0)),
        ],
        out_shape=[
            jax.ShapeDtypeStruct((NR, H), jnp.float32),
            jax.ShapeDtypeStruct((NR, 1), jnp.float32),
        ],
    )(agg, p, wrel, brel, wroot, pwc)


def _thresh_body(s_ref, alive_ref, t_ref, *, k):
    key = jnp.where(alive_ref[...] > 0, _ordkey(s_ref[...]), jnp.uint32(0))
    t = jnp.uint32(0)
    for b in range(31, -1, -1):
        cand = t | jnp.uint32(1 << b)
        c = jnp.sum((key >= cand).astype(jnp.int32))
        t = jnp.where(c >= k, cand, t)
    t_ref[...] = jnp.full((1, 1), t, jnp.uint32)


def _thresh(s, alive, k):
    return pl.pallas_call(
        functools.partial(_thresh_body, k=k),
        out_shape=jax.ShapeDtypeStruct((1, 1), jnp.uint32),
    )(s.reshape(NR // 128, 128), alive.reshape(NR // 128, 128))


def _pool_body(h_ref, s_ref, alive_ref, t_ref, p_ref, anew_ref, red_ref):
    i = pl.program_id(0)
    sv = s_ref[...]
    key = jnp.where(alive_ref[...] > 0, _ordkey(sv), jnp.uint32(0))
    kept = key >= t_ref[0, 0]
    pv = jnp.where(kept, h_ref[...] * jnp.tanh(sv), 0.0)
    p_ref[...] = pv
    anew_ref[...] = kept.astype(jnp.float32)
    bmx = jnp.max(jnp.where(kept, pv, -jnp.inf), axis=0, keepdims=True)
    bsm = jnp.sum(pv, axis=0, keepdims=True)

    @pl.when(i == 0)
    def _():
        red_ref[0:1, :] = bmx
        red_ref[1:2, :] = bsm

    @pl.when(i > 0)
    def _():
        red_ref[0:1, :] = jnp.maximum(red_ref[0:1, :], bmx)
        red_ref[1:2, :] = red_ref[1:2, :] + bsm


def _pool(h, s, alive, t):
    return pl.pallas_call(
        _pool_body,
        grid=(_NBLK,),
        in_specs=[
            pl.BlockSpec((_RB, H), lambda i: (i, 0)),
            pl.BlockSpec((_RB, 1), lambda i: (i, 0)),
            pl.BlockSpec((_RB, 1), lambda i: (i, 0)),
            pl.BlockSpec((1, 1), lambda i: (0, 0)),
        ],
        out_specs=[
            pl.BlockSpec((_RB, H), lambda i: (i, 0)),
            pl.BlockSpec((_RB, 1), lambda i: (i, 0)),
            pl.BlockSpec((2, H), lambda i: (0, 0)),
        ],
        out_shape=[
            jax.ShapeDtypeStruct((NR, H), jnp.float32),
            jax.ShapeDtypeStruct((NR, 1), jnp.float32),
            jax.ShapeDtypeStruct((2, H), jnp.float32),
        ],
    )(h, s, alive, t)


def _head_body(z_ref, wl1_ref, bl1_ref, g1_ref, be1_ref, wl2_ref, bl2_ref,
               g2_ref, be2_ref, wl3_ref, bl3_ref, out_ref):
    z = z_ref[...]
    sc = jnp.sqrt(jnp.float32(1.0 + EPS))
    fc1 = jnp.maximum(
        (jnp.dot(z, wl1_ref[...], preferred_element_type=jnp.float32)
         + bl1_ref[...]) / sc * g1_ref[...] + be1_ref[...], 0.0)
    fc2 = jnp.maximum(
        (jnp.dot(fc1, wl2_ref[...], preferred_element_type=jnp.float32)
         + bl2_ref[...]) / sc * g2_ref[...] + be2_ref[...], 0.0)
    logits = (jnp.dot(fc2, wl3_ref[...], preferred_element_type=jnp.float32)
              + bl3_ref[...])
    m = jnp.max(logits, axis=1, keepdims=True)
    e = jnp.exp(logits - m)
    out_ref[...] = e / jnp.sum(e, axis=1, keepdims=True)


def _head(z, Wl1, bl1, g1, be1, Wl2, bl2, g2, be2, Wl3, bl3):
    zp = jnp.zeros((8, 256), jnp.float32).at[0].set(z)
    wl2p = jnp.zeros((128, 128), jnp.float32).at[:, :64].set(Wl2)
    bl2p = jnp.full((1, 128), -1e30, jnp.float32).at[0, :64].set(bl2)
    g2p = jnp.zeros((1, 128), jnp.float32).at[0, :64].set(g2)
    be2p = jnp.zeros((1, 128), jnp.float32).at[0, :64].set(be2)
    wl3p = jnp.zeros((128, 128), jnp.float32).at[:64, :2].set(Wl3)
    bl3p = jnp.full((1, 128), -1e30, jnp.float32).at[0, :2].set(bl3)
    out = pl.pallas_call(
        _head_body,
        out_shape=jax.ShapeDtypeStruct((8, 128), jnp.float32),
    )(zp, Wl1, bl1[None, :], g1[None, :], be1[None, :], wl2p, bl2p, g2p,
      be2p, wl3p, bl3p)
    return out[0:1, 0:2]


def kernel(x, edge_index, batch, Wrel1, brel1, Wroot1, pw1, Wrel2, brel2,
           Wroot2, pw2, Wrel3, brel3, Wroot3, pw3, Wl1, bl1, Wl2, bl2, Wl3,
           bl3, g1, be1, g2, be2):
    src = jnp.concatenate([edge_index[0],
                           jnp.zeros((E2 - E,), jnp.int32)])
    dst = jnp.concatenate([edge_index[1],
                           jnp.full((E2 - E,), -1, jnp.int32)])
    bsrc, bdst, nwin = _bucket_call(src, dst)
    bsrc3 = bsrc.reshape(128, CAP_WB // 128, 128)
    bdst3 = bdst.reshape(128, CAP_WB // 128, 128)

    x_pad = jnp.zeros((NR, 16), jnp.float32).at[:N, :6].set(x)
    alive = jnp.zeros((NR, 1), jnp.float32).at[:N].set(1.0)
    wrel1p = jnp.zeros((16, H), jnp.float32).at[:6].set(Wrel1)
    wroot1p = jnp.zeros((16, H), jnp.float32).at[:6].set(Wroot1)

    layers = (
        (16, _conv16, wrel1p, brel1, wroot1p, pw1),
        (128, _conv128, Wrel2, brel2, Wroot2, pw2),
        (128, _conv128, Wrel3, brel3, Wroot3, pw3),
    )
    p = x_pad
    n_alive = N
    reads = []
    for (DP, conv, wrel, brel, wroot, pw) in layers:
        agg = conv(p, bsrc3, bdst3, nwin)
        h, s = _dense(agg, p, wrel, brel[None, :], wroot, pw[:, None], DP)
        k = int(math.ceil(RATIO * n_alive))
        t = _thresh(s, alive, k)
        p, alive, red = _pool(h, s, alive, t)
        reads.append(jnp.concatenate([red[0], red[1] / k]))
        n_alive = k
    z = reads[0] + reads[1] + reads[2]
    return _head(z, Wl1, bl1, g1, be1, Wl2, bl2, g2, be2, Wl3, bl3)
